# Initial kernel scaffold; baseline (speedup 1.0000x reference)
import jax
import jax.numpy as jnp
from jax import lax
from jax.experimental import pallas as pl
from jax.experimental.pallas import tpu as pltpu
from jax.experimental.pallas import tpu_sc as plsc

VOCAB = 1000000
D = 64
B = 16384
L = 50
N_IDX = B * L

NC = 2
NS = 16
NW = NC * NS
PER_W = N_IDX // NW

IDX_MINOR = 128
SUB = 8
CHUNK = SUB * IDX_MINOR
N_CHUNKS = PER_W // CHUNK

IDX_ROWS = N_IDX // IDX_MINOR
ROWS_PER_W = IDX_ROWS // NW


def _body(x_hbm, table_hbm, out_hbm, idx_v, rows_v, gsem):
    c = lax.axis_index("c")
    s = lax.axis_index("s")
    wid = s * NC + c

    def chunk(i, carry):
        irow = wid * ROWS_PER_W + i * SUB
        base = irow * IDX_MINOR
        pltpu.sync_copy(x_hbm.at[pl.ds(irow, SUB)], idx_v)
        cps = []
        for j in range(SUB):
            cps.append(pltpu.async_copy(
                table_hbm.at[idx_v.at[j]],
                rows_v.at[pl.ds(j * IDX_MINOR, IDX_MINOR)],
                gsem))
        for cp in cps:
            cp.wait()
        pltpu.sync_copy(rows_v, out_hbm.at[pl.ds(base, CHUNK)])
        return carry

    lax.fori_loop(0, N_CHUNKS, chunk, 0)


def kernel(x, table):
    x_flat = x.reshape(IDX_ROWS, IDX_MINOR).astype(jnp.int32)
    mesh = plsc.VectorSubcoreMesh(core_axis_name="c", subcore_axis_name="s")
    out = pl.kernel(
        _body,
        out_type=jax.ShapeDtypeStruct((N_IDX, D), jnp.float32),
        mesh=mesh,
        scratch_types=[
            pltpu.VMEM((SUB, IDX_MINOR), jnp.int32),
            pltpu.VMEM((CHUNK, D), jnp.float32),
            pltpu.SemaphoreType.DMA,
        ],
    )(x_flat, table)
    return out.reshape(B, L, D)


# speedup vs baseline: 1.8447x; 1.8447x over previous
import jax
import jax.numpy as jnp
from jax import lax
from jax.experimental import pallas as pl
from jax.experimental.pallas import tpu as pltpu
from jax.experimental.pallas import tpu_sc as plsc

VOCAB = 1000000
D = 64
B = 16384
L = 50
N_IDX = B * L

NC = 2
NS = 16
NW = NC * NS
PER_W = N_IDX // NW

IDX_MINOR = 128
SUB = 8
CHUNK = SUB * IDX_MINOR
N_CHUNKS = PER_W // CHUNK

IDX_ROWS = N_IDX // IDX_MINOR
ROWS_PER_W = IDX_ROWS // NW


def _body(x_hbm, table_hbm, out_hbm, idx_v, rows_v, gsem):
    c = lax.axis_index("c")
    s = lax.axis_index("s")
    wid = s * NC + c

    def chunk(i, carry):
        irow = wid * ROWS_PER_W + i * SUB
        base = irow * IDX_MINOR
        pltpu.sync_copy(x_hbm.at[pl.ds(irow, SUB)], idx_v)
        cps = []
        for j in range(SUB):
            cps.append(pltpu.async_copy(
                table_hbm.at[idx_v.at[j]],
                rows_v.at[pl.ds(j * IDX_MINOR, IDX_MINOR)],
                gsem))
        for cp in cps:
            cp.wait()
        pltpu.sync_copy(rows_v, out_hbm.at[pl.ds(base, CHUNK)])
        return carry

    lax.fori_loop(0, N_CHUNKS, chunk, 0)


def kernel(x, table):
    x_flat = x.reshape(IDX_ROWS, IDX_MINOR).astype(jnp.int32)
    mesh = plsc.VectorSubcoreMesh(core_axis_name="c", subcore_axis_name="s")
    out = pl.kernel(
        _body,
        out_type=jax.ShapeDtypeStruct((N_IDX, D), jnp.float32),
        mesh=mesh,
        scratch_types=[
            pltpu.VMEM((SUB, IDX_MINOR), jnp.int32),
            pltpu.VMEM((CHUNK, D), jnp.float32),
            pltpu.SemaphoreType.DMA,
        ],
        compiler_params=pltpu.CompilerParams(use_tc_tiling_on_sc=False),
    )(x_flat, table)
    return out.reshape(B, L, D)


# speedup vs baseline: 1.8627x; 1.0098x over previous
import jax
import jax.numpy as jnp
from jax import lax
from jax.experimental import pallas as pl
from jax.experimental.pallas import tpu as pltpu
from jax.experimental.pallas import tpu_sc as plsc

VOCAB = 1000000
D = 64
B = 16384
L = 50
N_IDX = B * L

NC = 2
NS = 16
NW = NC * NS
PER_W = N_IDX // NW

IDX_MINOR = 128
SUB = 5
CHUNK = SUB * IDX_MINOR
N_CHUNKS = PER_W // CHUNK
N_PAIRS = N_CHUNKS // 2

IDX_ROWS_W = PER_W // IDX_MINOR


def _fire_gathers(table_hbm, idx_all, rows_v, chunk_id, sem):
    cps = []
    for j in range(SUB):
        cps.append(pltpu.async_copy(
            table_hbm.at[idx_all.at[chunk_id * SUB + j]],
            rows_v.at[pl.ds(j * IDX_MINOR, IDX_MINOR)],
            sem))
    return cps


def _body(x_hbm, table_hbm, out_hbm, idx_all, rows_v0, rows_v1,
          gsem0, gsem1, wsem0, wsem1):
    c = lax.axis_index("c")
    s = lax.axis_index("s")
    wid = s * NC + c
    row0 = wid * IDX_ROWS_W
    base_w = wid * PER_W

    pltpu.sync_copy(x_hbm.at[pl.ds(row0, IDX_ROWS_W)], idx_all)

    def _wait_wb(rows_v, sem):
        pltpu.make_async_copy(rows_v, out_hbm.at[pl.ds(base_w, CHUNK)],
                              sem).wait()

    def step(p, carry):
        c0 = 2 * p
        c1 = 2 * p + 1

        @pl.when(p > 0)
        def _():
            _wait_wb(rows_v0, wsem0)
        g0 = _fire_gathers(table_hbm, idx_all, rows_v0, c0, gsem0)

        @pl.when(p > 0)
        def _():
            _wait_wb(rows_v1, wsem1)
        g1 = _fire_gathers(table_hbm, idx_all, rows_v1, c1, gsem1)

        for cp in g0:
            cp.wait()
        pltpu.async_copy(rows_v0, out_hbm.at[pl.ds(base_w + c0 * CHUNK, CHUNK)],
                         wsem0)

        for cp in g1:
            cp.wait()
        pltpu.async_copy(rows_v1, out_hbm.at[pl.ds(base_w + c1 * CHUNK, CHUNK)],
                         wsem1)
        return carry

    lax.fori_loop(0, N_PAIRS, step, 0)
    _wait_wb(rows_v0, wsem0)
    _wait_wb(rows_v1, wsem1)


def kernel(x, table):
    x_flat = x.reshape(N_IDX // IDX_MINOR, IDX_MINOR).astype(jnp.int32)
    mesh = plsc.VectorSubcoreMesh(core_axis_name="c", subcore_axis_name="s")
    out = pl.kernel(
        _body,
        out_type=jax.ShapeDtypeStruct((N_IDX, D), jnp.float32),
        mesh=mesh,
        scratch_types=[
            pltpu.VMEM((IDX_ROWS_W, IDX_MINOR), jnp.int32),
            pltpu.VMEM((CHUNK, D), jnp.float32),
            pltpu.VMEM((CHUNK, D), jnp.float32),
            pltpu.SemaphoreType.DMA,
            pltpu.SemaphoreType.DMA,
            pltpu.SemaphoreType.DMA,
            pltpu.SemaphoreType.DMA,
        ],
        compiler_params=pltpu.CompilerParams(use_tc_tiling_on_sc=False),
    )(x_flat, table)
    return out.reshape(B, L, D)


# speedup vs baseline: 1.8712x; 1.0046x over previous
import jax
import jax.numpy as jnp
from jax import lax
from jax.experimental import pallas as pl
from jax.experimental.pallas import tpu as pltpu
from jax.experimental.pallas import tpu_sc as plsc

VOCAB = 1000000
D = 64
B = 16384
L = 50
N_IDX = B * L

NC = 2
NS = 16
NW = NC * NS
B_PER_W = B // NW

B_CHUNK = 16
CHUNK = B_CHUNK * L
N_CHUNKS = B_PER_W // B_CHUNK
N_PAIRS = N_CHUNKS // 2

GATHER_SIZES = [128] * (CHUNK // 128) + ([CHUNK % 128] if CHUNK % 128 else [])


def _fire_gathers(table_hbm, xv, rows_v, sem):
    cps = []
    off = 0
    for n in GATHER_SIZES:
        cps.append(pltpu.async_copy(
            table_hbm.at[xv.at[pl.ds(off, n)]],
            rows_v.at[pl.ds(off, n)],
            sem))
        off += n
    return cps


def _body(x_hbm, table_hbm, out_hbm, xv0, xv1, rows_v0, rows_v1,
          gsem0, gsem1, wsem0, wsem1):
    c = lax.axis_index("c")
    s = lax.axis_index("s")
    wid = s * NC + c
    b0w = wid * B_PER_W

    def _fire_wb(rows_v, b_base, sem):
        for bb in range(B_CHUNK):
            pltpu.async_copy(rows_v.at[pl.ds(bb * L, L)],
                             out_hbm.at[b_base + bb], sem)

    def _wait_wb(rows_v, sem):
        for bb in range(B_CHUNK):
            pltpu.make_async_copy(rows_v.at[pl.ds(bb * L, L)],
                                  out_hbm.at[b0w + bb], sem).wait()

    def step(p, carry):
        b0 = b0w + 2 * p * B_CHUNK
        b1 = b0 + B_CHUNK
        n0 = b0 * L
        n1 = n0 + CHUNK

        @pl.when(p > 0)
        def _():
            _wait_wb(rows_v0, wsem0)
        pltpu.sync_copy(x_hbm.at[pl.ds(n0, CHUNK)], xv0)
        g0 = _fire_gathers(table_hbm, xv0, rows_v0, gsem0)

        @pl.when(p > 0)
        def _():
            _wait_wb(rows_v1, wsem1)
        pltpu.sync_copy(x_hbm.at[pl.ds(n1, CHUNK)], xv1)
        g1 = _fire_gathers(table_hbm, xv1, rows_v1, gsem1)

        for cp in g0:
            cp.wait()
        _fire_wb(rows_v0, b0, wsem0)

        for cp in g1:
            cp.wait()
        _fire_wb(rows_v1, b1, wsem1)
        return carry

    lax.fori_loop(0, N_PAIRS, step, 0)
    _wait_wb(rows_v0, wsem0)
    _wait_wb(rows_v1, wsem1)


def kernel(x, table):
    x_flat = x.reshape(N_IDX).astype(jnp.int32)
    mesh = plsc.VectorSubcoreMesh(core_axis_name="c", subcore_axis_name="s")
    out = pl.kernel(
        _body,
        out_type=jax.ShapeDtypeStruct((B, L, D), jnp.float32),
        mesh=mesh,
        scratch_types=[
            pltpu.VMEM((CHUNK,), jnp.int32),
            pltpu.VMEM((CHUNK,), jnp.int32),
            pltpu.VMEM((CHUNK, D), jnp.float32),
            pltpu.VMEM((CHUNK, D), jnp.float32),
            pltpu.SemaphoreType.DMA,
            pltpu.SemaphoreType.DMA,
            pltpu.SemaphoreType.DMA,
            pltpu.SemaphoreType.DMA,
        ],
        compiler_params=pltpu.CompilerParams(use_tc_tiling_on_sc=False),
    )(x_flat, table)
    return out
